# Initial kernel scaffold; baseline (speedup 1.0000x reference)
import jax
import jax.numpy as jnp
from jax.experimental import pallas as pl


def kernel(quantum_states):
    raise NotImplementedError("write your pallas kernel here")


# speedup vs baseline: 3.8339x; 3.8339x over previous
import jax
import jax.numpy as jnp
from jax.experimental import pallas as pl
from jax.experimental.pallas import tpu as pltpu

_THRESHOLD = 0.95
_SECONDARY = 0.5


def _lap_block(states_ref, states_t_ref, out_ref):
    s = states_ref[0]
    st = states_t_ref[0]
    gram = jax.lax.dot_general(
        s, st, (((1,), (0,)), ((), ())), preferred_element_type=jnp.float32)
    fid = gram * gram
    w = jnp.where(fid >= _THRESHOLD, jnp.float32(1.0),
                  jnp.where(fid >= _SECONDARY, jnp.float32(_SECONDARY),
                            jnp.float32(0.0)))
    row = jax.lax.broadcasted_iota(jnp.int32, w.shape, 0)
    col = jax.lax.broadcasted_iota(jnp.int32, w.shape, 1)
    diag = row == col
    w = jnp.where(diag, jnp.float32(0.0), w)
    deg = jnp.sum(w, axis=1, keepdims=True)
    out_ref[0] = jnp.where(diag, deg, -w)


def kernel(quantum_states):
    batch, num_states, n_wires = quantum_states.shape
    states_t = jnp.swapaxes(quantum_states, 1, 2)
    return pl.pallas_call(
        _lap_block,
        grid=(batch,),
        in_specs=[
            pl.BlockSpec((1, num_states, n_wires), lambda b: (b, 0, 0)),
            pl.BlockSpec((1, n_wires, num_states), lambda b: (b, 0, 0)),
        ],
        out_specs=pl.BlockSpec((1, num_states, num_states), lambda b: (b, 0, 0)),
        out_shape=jax.ShapeDtypeStruct((batch, num_states, num_states),
                                       jnp.float32),
        compiler_params=pltpu.CompilerParams(
            dimension_semantics=("parallel",)),
    )(quantum_states, states_t)
